# Initial kernel scaffold; baseline (speedup 1.0000x reference)
import jax
import jax.numpy as jnp
from jax.experimental import pallas as pl


def kernel(nodes, pivots, weight_tensor):
    raise NotImplementedError("write your pallas kernel here")


# speedup vs baseline: 37.7783x; 37.7783x over previous
import functools

import jax
import jax.numpy as jnp
from jax.experimental import pallas as pl
from jax.experimental.pallas import tpu as pltpu

_NUM_PERS = 4
_D = 64
_TOPK = 16
_NEG = -3.0


def _normalize_feats(x, w):
    feats = []
    for p in range(_NUM_PERS):
        xp = x * w[p][None, :]
        norm = jnp.sqrt(jnp.sum(xp * xp, axis=1, keepdims=True))
        inv = 1.0 / jnp.maximum(norm, 1e-12)
        feats.append((xp * inv).astype(jnp.bfloat16))
    return jnp.concatenate(feats, axis=1)


def _block_kernel(nodes_ref, pivots_ref, w_ref, out_ref, pfeat_ref):
    pid = pl.program_id(0)

    @pl.when(pid == 0)
    def _():
        pfeat_ref[...] = _normalize_feats(pivots_ref[...], w_ref[...])

    nfeat = _normalize_feats(nodes_ref[...], w_ref[...])
    scores = jax.lax.dot_general(
        nfeat, pfeat_ref[...],
        dimension_numbers=(((1,), (1,)), ((), ())),
        preferred_element_type=jnp.float32,
    ) * 0.25

    b = scores
    for _ in range(_TOPK):
        m = jnp.max(b, axis=1, keepdims=True)
        b = jnp.where(b == m, _NEG, b)
    out_ref[...] = jnp.where(b == _NEG, scores, 0.0)


@jax.jit
def kernel(nodes, pivots, weight_tensor):
    n, d = nodes.shape
    m = pivots.shape[0]
    bn = 400
    grid = n // bn
    return pl.pallas_call(
        _block_kernel,
        grid=(grid,),
        in_specs=[
            pl.BlockSpec((bn, d), lambda i: (i, 0)),
            pl.BlockSpec((m, d), lambda i: (0, 0)),
            pl.BlockSpec((_NUM_PERS, d), lambda i: (0, 0)),
        ],
        out_specs=pl.BlockSpec((bn, m), lambda i: (i, 0)),
        out_shape=jax.ShapeDtypeStruct((n, m), jnp.float32),
        scratch_shapes=[pltpu.VMEM((m, _NUM_PERS * d), jnp.bfloat16)],
    )(nodes, pivots, weight_tensor)
